# baseline (speedup 1.0000x reference)
import functools

import jax
import jax.numpy as jnp
from jax import lax
from jax.experimental import pallas as pl
from jax.experimental.pallas import tpu as pltpu
from jax.experimental.pallas import tpu_sc as plsc

BATCH = 16384
DIM = 1024
WPD = DIM // 4

_info = plsc.get_sparse_core_info()
_NC, _NS, _L = _info.num_cores, _info.num_subcores, _info.num_lanes
NW = _NC * _NS
ROWS_PER_W = BATCH // NW
R = 16
NCHUNK = ROWS_PER_W // R


def _body(x_hbm, m_hbm, perm_hbm, widx_hbm, wsh_hbm, xo_hbm, mo_hbm,
          perm_v, widx_v, wsh_v,
          xin0, xin1, xout0, xout1, min0, min1, mout0, mout1,
          sxi0, sxi1, smi0, smi1, sxo0, sxo1, smo0, smo1):
    wid = lax.axis_index("s") * _NC + lax.axis_index("c")
    row_base = wid * ROWS_PER_W

    pltpu.sync_copy(perm_hbm, perm_v)
    pltpu.sync_copy(widx_hbm, widx_v)
    pltpu.sync_copy(wsh_hbm, wsh_v)

    bufs = ((xin0, min0, xout0, mout0, sxi0, smi0, sxo0, smo0),
            (xin1, min1, xout1, mout1, sxi1, smi1, sxo1, smo1))

    def issue_in(ci, k):
        xin, min_, _, _, sxi, smi, _, _ = bufs[k]
        e0 = (row_base + ci * R) * DIM
        w0 = (row_base + ci * R) * WPD
        pltpu.async_copy(x_hbm.at[pl.ds(e0, R * DIM)], xin, sxi)
        pltpu.async_copy(m_hbm.at[pl.ds(w0, R * WPD)], min_, smi)

    def wait_in(k):
        xin, min_, _, _, sxi, smi, _, _ = bufs[k]
        pltpu.make_async_copy(x_hbm.at[pl.ds(0, R * DIM)], xin, sxi).wait()
        pltpu.make_async_copy(m_hbm.at[pl.ds(0, R * WPD)], min_, smi).wait()

    def issue_out(ci, k):
        _, _, xout, mout, _, _, sxo, smo = bufs[k]
        e0 = (row_base + ci * R) * DIM
        w0 = (row_base + ci * R) * WPD
        pltpu.async_copy(xout, xo_hbm.at[pl.ds(e0, R * DIM)], sxo)
        pltpu.async_copy(mout, mo_hbm.at[pl.ds(w0, R * WPD)], smo)

    def wait_out(k):
        _, _, xout, mout, _, _, sxo, smo = bufs[k]
        pltpu.make_async_copy(xout, xo_hbm.at[pl.ds(0, R * DIM)], sxo).wait()
        pltpu.make_async_copy(mout, mo_hbm.at[pl.ds(0, R * WPD)], smo).wait()

    def compute(k):
        xin, min_, xout, mout, _, _, _, _ = bufs[k]

        def xg_body(g, carry):
            idx = perm_v[pl.ds(g * _L, _L)]
            vals = [plsc.load_gather(xin, [idx + r * DIM]) for r in range(R)]
            for r in range(R):
                xout[pl.ds(g * _L + r * DIM, _L)] = vals[r]
            return carry
        lax.fori_loop(0, DIM // _L, xg_body, 0, unroll=1)

        def mg_body(g, carry):
            wi = [widx_v[pl.ds(c * WPD + g * _L, _L)] for c in range(4)]
            sh = [wsh_v[pl.ds(c * WPD + g * _L, _L)] for c in range(4)]
            for r in range(R):
                out = None
                for c in range(4):
                    w = plsc.load_gather(min_, [wi[c] + r * WPD])
                    b = lax.shift_left(
                        lax.shift_right_logical(w, sh[c]) & 0xFF, 8 * c)
                    out = b if c == 0 else (out | b)
                mout[pl.ds(g * _L + r * WPD, _L)] = out
            return carry
        lax.fori_loop(0, WPD // _L, mg_body, 0, unroll=1)

    issue_in(0, 0)
    issue_in(1, 1)
    for ci in range(2):
        wait_in(ci)
        compute(ci)
        issue_out(ci, ci)
        issue_in(ci + 2, ci)

    def outer(it, carry):
        cb = 2 + it * 2
        for k in range(2):
            ci = cb + k
            wait_in(k)
            wait_out(k)
            compute(k)
            issue_out(ci, k)

            @pl.when(ci + 2 < NCHUNK)
            def _():
                issue_in(ci + 2, k)
        return carry
    lax.fori_loop(0, (NCHUNK - 2) // 2, outer, 0, unroll=1)

    wait_out(0)
    wait_out(1)


_mesh = plsc.VectorSubcoreMesh(core_axis_name="c", subcore_axis_name="s")

_sc_call = functools.partial(
    pl.kernel,
    out_type=(
        jax.ShapeDtypeStruct((BATCH * DIM,), jnp.float32),
        jax.ShapeDtypeStruct((BATCH * WPD,), jnp.int32),
    ),
    mesh=_mesh,
    compiler_params=pltpu.CompilerParams(needs_layout_passes=False),
    scratch_types=[
        pltpu.VMEM((DIM,), jnp.int32),
        pltpu.VMEM((4 * WPD,), jnp.int32),
        pltpu.VMEM((4 * WPD,), jnp.int32),
        pltpu.VMEM((R * DIM,), jnp.float32),
        pltpu.VMEM((R * DIM,), jnp.float32),
        pltpu.VMEM((R * DIM,), jnp.float32),
        pltpu.VMEM((R * DIM,), jnp.float32),
        pltpu.VMEM((R * WPD,), jnp.int32),
        pltpu.VMEM((R * WPD,), jnp.int32),
        pltpu.VMEM((R * WPD,), jnp.int32),
        pltpu.VMEM((R * WPD,), jnp.int32),
    ] + [pltpu.SemaphoreType.DMA] * 8,
)


def kernel(x, observed_mask, perm, inv_perm):
    del inv_perm
    m_words = lax.bitcast_convert_type(
        observed_mask.reshape(BATCH, WPD, 4).view(jnp.uint8), jnp.int32)
    pj = perm.reshape(WPD, 4)
    widx = jnp.transpose(pj >> 2).reshape(4 * WPD).astype(jnp.int32)
    wsh = jnp.transpose((pj & 3) << 3).reshape(4 * WPD).astype(jnp.int32)

    xo, mo = _sc_call(_body)(
        x.reshape(BATCH * DIM), m_words.reshape(BATCH * WPD),
        perm, widx, wsh)
    x_out = xo.reshape(BATCH, DIM)
    m_out = lax.bitcast_convert_type(mo.reshape(BATCH, WPD), jnp.uint8)
    m_out = m_out.reshape(BATCH, DIM) != 0
    return (x_out, m_out)


# speedup vs baseline: 3.6734x; 3.6734x over previous
import functools

import jax
import jax.numpy as jnp
from jax import lax
from jax.experimental import pallas as pl
from jax.experimental.pallas import tpu as pltpu
from jax.experimental.pallas import tpu_sc as plsc

BATCH = 16384
DIM = 1024

_info = plsc.get_sparse_core_info()
_NC, _NS, _L = _info.num_cores, _info.num_subcores, _info.num_lanes
NW = _NC * _NS
ROWS_PER_W = BATCH // NW
R = 16
NCHUNK = ROWS_PER_W // R


def _x_body(x_hbm, perm_hbm, xo_hbm,
            perm_v, xin0, xin1, xout0, xout1, sxi0, sxi1, sxo0, sxo1):
    wid = lax.axis_index("s") * _NC + lax.axis_index("c")
    row_base = wid * ROWS_PER_W

    pltpu.sync_copy(perm_hbm, perm_v)

    bufs = ((xin0, xout0, sxi0, sxo0), (xin1, xout1, sxi1, sxo1))

    def issue_in(ci, k):
        xin, _, sxi, _ = bufs[k]
        pltpu.async_copy(x_hbm.at[pl.ds(row_base + ci * R, R)], xin, sxi)

    def wait_in(k):
        xin, _, sxi, _ = bufs[k]
        pltpu.make_async_copy(x_hbm.at[pl.ds(0, R)], xin, sxi).wait()

    def issue_out(ci, k):
        _, xout, _, sxo = bufs[k]
        pltpu.async_copy(xout, xo_hbm.at[pl.ds(row_base + ci * R, R)], sxo)

    def wait_out(k):
        _, xout, _, sxo = bufs[k]
        pltpu.make_async_copy(xout, xo_hbm.at[pl.ds(0, R)], sxo).wait()

    def compute(k):
        xin, xout, _, _ = bufs[k]

        def xg_body(g, carry):
            idx = perm_v[pl.ds(g * _L, _L)]
            for r in range(R):
                rv = jnp.full((_L,), r, jnp.int32)
                v = plsc.load_gather(xin, [rv, idx])
                xout[r, pl.ds(g * _L, _L)] = v
            return carry
        lax.fori_loop(0, DIM // _L, xg_body, 0, unroll=1)

    issue_in(0, 0)
    issue_in(1, 1)
    for ci in range(2):
        wait_in(ci)
        compute(ci)
        issue_out(ci, ci)
        issue_in(ci + 2, ci)

    def outer(it, carry):
        cb = 2 + it * 2
        for k in range(2):
            ci = cb + k
            wait_in(k)
            wait_out(k)
            compute(k)
            issue_out(ci, k)

            @pl.when(ci + 2 < NCHUNK)
            def _():
                issue_in(ci + 2, k)
        return carry
    lax.fori_loop(0, (NCHUNK - 2) // 2, outer, 0, unroll=1)

    wait_out(0)
    wait_out(1)


_mesh = plsc.VectorSubcoreMesh(core_axis_name="c", subcore_axis_name="s")

_x_call = functools.partial(
    pl.kernel,
    out_type=jax.ShapeDtypeStruct((BATCH, DIM), jnp.float32),
    mesh=_mesh,
    compiler_params=pltpu.CompilerParams(needs_layout_passes=False),
    scratch_types=[
        pltpu.VMEM((DIM,), jnp.int32),
        pltpu.VMEM((R, DIM), jnp.float32),
        pltpu.VMEM((R, DIM), jnp.float32),
        pltpu.VMEM((R, DIM), jnp.float32),
        pltpu.VMEM((R, DIM), jnp.float32),
    ] + [pltpu.SemaphoreType.DMA] * 4,
)


MROWS = 1024


def _mask_body(perm_ref, m_ref, out_ref, p_scratch):
    @pl.when(pl.program_id(0) == 0)
    def _():
        iota = lax.broadcasted_iota(jnp.int32, (DIM, DIM), 0)
        p_scratch[...] = (iota == perm_ref[0][None, :]).astype(jnp.bfloat16)

    m = m_ref[...].astype(jnp.bfloat16)
    acc = jnp.dot(m, p_scratch[...], preferred_element_type=jnp.float32)
    out_ref[...] = acc > 0.5


_mask_call = pl.pallas_call(
    _mask_body,
    grid=(BATCH // MROWS,),
    in_specs=[
        pl.BlockSpec((1, DIM), lambda i: (0, 0)),
        pl.BlockSpec((MROWS, DIM), lambda i: (i, 0)),
    ],
    out_specs=pl.BlockSpec((MROWS, DIM), lambda i: (i, 0)),
    out_shape=jax.ShapeDtypeStruct((BATCH, DIM), jnp.bool_),
    scratch_shapes=[pltpu.VMEM((DIM, DIM), jnp.bfloat16)],
)


def kernel(x, observed_mask, perm, inv_perm):
    del inv_perm
    x_out = _x_call(_x_body)(x, perm)
    m_out = _mask_call(perm.reshape(1, DIM), observed_mask)
    return (x_out, m_out)


# speedup vs baseline: 4.3630x; 1.1877x over previous
import functools

import jax
import jax.numpy as jnp
from jax import lax
from jax.experimental import pallas as pl
from jax.experimental.pallas import tpu as pltpu
from jax.experimental.pallas import tpu_sc as plsc

BATCH = 16384
DIM = 1024

_info = plsc.get_sparse_core_info()
_NC, _NS, _L = _info.num_cores, _info.num_subcores, _info.num_lanes
NW = _NC * _NS
ROWS_PER_W = BATCH // NW
R = 16
NCHUNK = ROWS_PER_W // R


def _x_body(x_hbm, perm_hbm, xo_hbm,
            perm_v, xin0, xin1, xout0, xout1, sxi0, sxi1, sxo0, sxo1):
    wid = lax.axis_index("s") * _NC + lax.axis_index("c")
    row_base = wid * ROWS_PER_W

    pltpu.sync_copy(perm_hbm, perm_v)

    bufs = ((xin0, xout0, sxi0, sxo0), (xin1, xout1, sxi1, sxo1))

    def issue_in(ci, k):
        xin, _, sxi, _ = bufs[k]
        pltpu.async_copy(x_hbm.at[pl.ds(row_base + ci * R, R)], xin, sxi)

    def wait_in(k):
        xin, _, sxi, _ = bufs[k]
        pltpu.make_async_copy(x_hbm.at[pl.ds(0, R)], xin, sxi).wait()

    def issue_out(ci, k):
        _, xout, _, sxo = bufs[k]
        pltpu.async_copy(xout, xo_hbm.at[pl.ds(row_base + ci * R, R)], sxo)

    def wait_out(k):
        _, xout, _, sxo = bufs[k]
        pltpu.make_async_copy(xout, xo_hbm.at[pl.ds(0, R)], sxo).wait()

    def compute(k):
        xin, xout = bufs[k][0], bufs[k][1]
        rows = [jnp.full((_L,), r, jnp.int32) for r in range(R)]

        @plsc.parallel_loop(0, DIM // _L, unroll=2)
        def _(g):
            cv = perm_v[pl.ds(g * _L, _L)]
            for r in range(R):
                v = plsc.load_gather(xin, [rows[r], cv])
                xout[r, pl.ds(g * _L, _L)] = v

    issue_in(0, 0)
    issue_in(1, 1)
    for ci in range(2):
        wait_in(ci)
        compute(ci)
        issue_out(ci, ci)
        issue_in(ci + 2, ci)

    def outer(it, carry):
        cb = 2 + it * 2
        for k in range(2):
            ci = cb + k
            wait_in(k)
            wait_out(k)
            compute(k)
            issue_out(ci, k)

            @pl.when(ci + 2 < NCHUNK)
            def _():
                issue_in(ci + 2, k)
        return carry
    lax.fori_loop(0, (NCHUNK - 2) // 2, outer, 0, unroll=1)

    wait_out(0)
    wait_out(1)


_mesh = plsc.VectorSubcoreMesh(core_axis_name="c", subcore_axis_name="s")

_x_call = functools.partial(
    pl.kernel,
    out_type=jax.ShapeDtypeStruct((BATCH, DIM), jnp.float32),
    mesh=_mesh,
    compiler_params=pltpu.CompilerParams(needs_layout_passes=False),
    scratch_types=[
        pltpu.VMEM((DIM,), jnp.int32),
        pltpu.VMEM((R, DIM), jnp.float32),
        pltpu.VMEM((R, DIM), jnp.float32),
        pltpu.VMEM((R, DIM), jnp.float32),
        pltpu.VMEM((R, DIM), jnp.float32),
    ] + [pltpu.SemaphoreType.DMA] * 4,
)


MROWS = 1024


def _mask_body(perm_ref, m_ref, out_ref, p_scratch):
    @pl.when(pl.program_id(0) == 0)
    def _():
        iota = lax.broadcasted_iota(jnp.int32, (DIM, DIM), 0)
        p_scratch[...] = (iota == perm_ref[0][None, :]).astype(jnp.bfloat16)

    m = m_ref[...].astype(jnp.bfloat16)
    acc = jnp.dot(m, p_scratch[...], preferred_element_type=jnp.float32)
    out_ref[...] = acc > 0.5


_mask_call = pl.pallas_call(
    _mask_body,
    grid=(BATCH // MROWS,),
    in_specs=[
        pl.BlockSpec((1, DIM), lambda i: (0, 0)),
        pl.BlockSpec((MROWS, DIM), lambda i: (i, 0)),
    ],
    out_specs=pl.BlockSpec((MROWS, DIM), lambda i: (i, 0)),
    out_shape=jax.ShapeDtypeStruct((BATCH, DIM), jnp.bool_),
    scratch_shapes=[pltpu.VMEM((DIM, DIM), jnp.bfloat16)],
)


def kernel(x, observed_mask, perm, inv_perm):
    del inv_perm
    x_out = _x_call(_x_body)(x, perm)
    m_out = _mask_call(perm.reshape(1, DIM), observed_mask)
    return (x_out, m_out)


# speedup vs baseline: 4.3638x; 1.0002x over previous
import functools

import jax
import jax.numpy as jnp
from jax import lax
from jax.experimental import pallas as pl
from jax.experimental.pallas import tpu as pltpu
from jax.experimental.pallas import tpu_sc as plsc

BATCH = 16384
DIM = 1024

_info = plsc.get_sparse_core_info()
_NC, _NS, _L = _info.num_cores, _info.num_subcores, _info.num_lanes
NW = _NC * _NS
ROWS_PER_W = BATCH // NW
R = 8
NBUF = 4
NCHUNK = ROWS_PER_W // R


def _x_body(x_hbm, perm_hbm, xo_hbm, perm_v, *rest):
    xins = rest[0:NBUF]
    xouts = rest[NBUF:2 * NBUF]
    sxis = rest[2 * NBUF:3 * NBUF]
    sxos = rest[3 * NBUF:4 * NBUF]

    wid = lax.axis_index("s") * _NC + lax.axis_index("c")
    row_base = wid * ROWS_PER_W

    pltpu.sync_copy(perm_hbm, perm_v)

    def issue_in(ci, k):
        pltpu.async_copy(x_hbm.at[pl.ds(row_base + ci * R, R)],
                         xins[k], sxis[k])

    def wait_in(k):
        pltpu.make_async_copy(x_hbm.at[pl.ds(0, R)], xins[k], sxis[k]).wait()

    def issue_out(ci, k):
        pltpu.async_copy(xouts[k], xo_hbm.at[pl.ds(row_base + ci * R, R)],
                         sxos[k])

    def wait_out(k):
        pltpu.make_async_copy(xouts[k], xo_hbm.at[pl.ds(0, R)],
                              sxos[k]).wait()

    rows = [jnp.full((_L,), r, jnp.int32) for r in range(R)]

    def compute(k):
        xin, xout = xins[k], xouts[k]

        @plsc.parallel_loop(0, DIM // _L, unroll=2)
        def _(g):
            cv = perm_v[pl.ds(g * _L, _L)]
            for r in range(R):
                v = plsc.load_gather(xin, [rows[r], cv])
                xout[r, pl.ds(g * _L, _L)] = v

    for ci in range(NBUF):
        issue_in(ci, ci)
    for ci in range(NBUF):
        wait_in(ci)
        compute(ci)
        issue_out(ci, ci)
        issue_in(ci + NBUF, ci)

    def outer(it, carry):
        cb = NBUF + it * NBUF
        for k in range(NBUF):
            ci = cb + k
            wait_in(k)
            wait_out(k)
            compute(k)
            issue_out(ci, k)

            @pl.when(ci + NBUF < NCHUNK)
            def _():
                issue_in(ci + NBUF, k)
        return carry
    lax.fori_loop(0, (NCHUNK - NBUF) // NBUF, outer, 0, unroll=1)

    for k in range(NBUF):
        wait_out(k)


_mesh = plsc.VectorSubcoreMesh(core_axis_name="c", subcore_axis_name="s")

_x_call = functools.partial(
    pl.kernel,
    out_type=jax.ShapeDtypeStruct((BATCH, DIM), jnp.float32),
    mesh=_mesh,
    compiler_params=pltpu.CompilerParams(needs_layout_passes=False),
    scratch_types=(
        [pltpu.VMEM((DIM,), jnp.int32)]
        + [pltpu.VMEM((R, DIM), jnp.float32)] * (2 * NBUF)
        + [pltpu.SemaphoreType.DMA] * (2 * NBUF)
    ),
)


MROWS = 1024


def _mask_body(perm_ref, m_ref, out_ref, p_scratch):
    @pl.when(pl.program_id(0) == 0)
    def _():
        iota = lax.broadcasted_iota(jnp.int32, (DIM, DIM), 0)
        p_scratch[...] = (iota == perm_ref[0][None, :]).astype(jnp.bfloat16)

    m = m_ref[...].astype(jnp.bfloat16)
    acc = jnp.dot(m, p_scratch[...], preferred_element_type=jnp.float32)
    out_ref[...] = acc > 0.5


_mask_call = pl.pallas_call(
    _mask_body,
    grid=(BATCH // MROWS,),
    in_specs=[
        pl.BlockSpec((1, DIM), lambda i: (0, 0)),
        pl.BlockSpec((MROWS, DIM), lambda i: (i, 0)),
    ],
    out_specs=pl.BlockSpec((MROWS, DIM), lambda i: (i, 0)),
    out_shape=jax.ShapeDtypeStruct((BATCH, DIM), jnp.bool_),
    scratch_shapes=[pltpu.VMEM((DIM, DIM), jnp.bfloat16)],
)


def kernel(x, observed_mask, perm, inv_perm):
    del inv_perm
    x_out = _x_call(_x_body)(x, perm)
    m_out = _mask_call(perm.reshape(1, DIM), observed_mask)
    return (x_out, m_out)
